# baseline (speedup 1.0000x reference)
import functools

import jax
import jax.numpy as jnp
from jax import lax
from jax.experimental import pallas as pl
from jax.experimental.pallas import tpu as pltpu
from jax.experimental.pallas import tpu_sc as plsc

F = 26
V = 100000
E = 16
A = 16
B = 4096

NC = 2
NS = 16
NW = NC * NS
N = B * F
N_PER_W = N // NW
CH = 128
N_CH = N_PER_W // CH
BS = B // NW

BT = 256
FP = 32


@functools.lru_cache(maxsize=None)
def _get_sc_gather():
    mesh = plsc.VectorSubcoreMesh(core_axis_name="c", subcore_axis_name="s")

    @functools.partial(
        pl.kernel,
        mesh=mesh,
        compiler_params=pltpu.CompilerParams(use_tc_tiling_on_sc=False),
        out_type=jax.ShapeDtypeStruct((B, F * E), jnp.float32),
        scratch_types=[
            pltpu.VMEM((F, BS), jnp.int32),
            pltpu.VMEM((F, BS, E), jnp.float32),
            pltpu.SemaphoreType.DMA,
            pltpu.SemaphoreType.DMA,
        ],
    )
    def _sc_gather(tbl_hbm, idx_hbm, rows_out, idx_v, buf, sem, sem2):
        wid = lax.axis_index("s") * NC + lax.axis_index("c")
        base = wid * BS
        pltpu.sync_copy(idx_hbm.at[wid], idx_v)
        cps = []
        for f in range(F):
            cps.append(pltpu.async_copy(
                tbl_hbm.at[idx_v.at[f]], buf.at[f], sem))
        outs = []
        for f in range(F):
            cps[f].wait()
            outs.append(pltpu.async_copy(
                buf.at[f],
                rows_out.at[pl.ds(base, BS), pl.ds(f * E, E)], sem2))
        for cp in outs:
            cp.wait()

    return _sc_gather


@functools.lru_cache(maxsize=None)
def _get_sc_gather_e1():
    mesh = plsc.VectorSubcoreMesh(core_axis_name="c", subcore_axis_name="s")

    @functools.partial(
        pl.kernel,
        mesh=mesh,
        compiler_params=pltpu.CompilerParams(use_tc_tiling_on_sc=False),
        out_type=jax.ShapeDtypeStruct((N,), jnp.float32),
        scratch_types=[
            pltpu.VMEM((N_PER_W,), jnp.int32),
            pltpu.VMEM((N_PER_W,), jnp.float32),
            pltpu.SemaphoreType.DMA,
        ],
    )
    def _sc_gather_e1(emb1_hbm, idx_hbm, e1_out, idx_v, e1_v, sem):
        wid = lax.axis_index("s") * NC + lax.axis_index("c")
        base = wid * N_PER_W
        pltpu.sync_copy(idx_hbm.at[wid], idx_v)
        pltpu.async_copy(emb1_hbm.at[idx_v], e1_v, sem).wait()
        pltpu.sync_copy(e1_v, e1_out.at[pl.ds(base, N_PER_W)])

    return _sc_gather_e1


def _tr(m):
    chunks = []
    L = m.shape[1]
    for c in range(0, L, 128):
        w = min(128, L - c)
        chunks.append(jnp.transpose(m[:, c:c + w]))
    return jnp.concatenate(chunks, axis=0) if len(chunks) > 1 else chunks[0]


def _tc_body(rows_ref, xvb_ref, e1b_ref, w1t_ref, h_ref, p_ref, b_ref, out_ref):
    w_col = jnp.sum(w1t_ref[...] * h_ref[...], axis=1, keepdims=True)
    p_col = p_ref[...]
    zpad = jnp.zeros((BT, FP - F), jnp.float32)
    xv = _tr(jnp.concatenate([xvb_ref[...], zpad], axis=1))
    e1t = _tr(jnp.concatenate([e1b_ref[...], zpad], axis=1))
    first = jnp.sum(e1t * xv, axis=0, keepdims=True)
    t = _tr(rows_ref[...])
    s_all = jnp.concatenate(
        [t[f * E:(f + 1) * E, :] * xv[f:f + 1, :] for f in range(F)], axis=0)
    num = jnp.zeros_like(first)
    den = jnp.zeros_like(first)
    for i in range(F - 1):
        nj = F - 1 - i
        si = s_all[i * E:(i + 1) * E, :]
        swi = si * w_col
        spi = si * p_col
        rest = s_all[(i + 1) * E:, :]
        gw = jnp.sum((rest * jnp.tile(swi, (nj, 1))).reshape(nj, E, BT), axis=1)
        gp = jnp.sum((rest * jnp.tile(spi, (nj, 1))).reshape(nj, E, BT), axis=1)
        ew = jnp.exp(gw)
        den = den + jnp.sum(ew, axis=0, keepdims=True)
        num = num + jnp.sum(gp * ew, axis=0, keepdims=True)
    out_ref[...] = b_ref[...] + first + num / den


_tc_compute = pl.pallas_call(
    _tc_body,
    grid=(B // BT,),
    in_specs=[
        pl.BlockSpec((BT, F * E), lambda i: (i, 0)),
        pl.BlockSpec((BT, F), lambda i: (i, 0)),
        pl.BlockSpec((BT, F), lambda i: (i, 0)),
        pl.BlockSpec((E, A), lambda i: (0, 0)),
        pl.BlockSpec((1, A), lambda i: (0, 0)),
        pl.BlockSpec((E, 1), lambda i: (0, 0)),
        pl.BlockSpec((1, 1), lambda i: (0, 0)),
    ],
    out_specs=pl.BlockSpec((1, BT), lambda i: (0, i)),
    out_shape=jax.ShapeDtypeStruct((1, B), jnp.float32),
)


def kernel(Xi, Xv, emb1, emb2, W1, b1, H, P, bias):
    del b1
    idx = Xi[:, :, 0].astype(jnp.int32)
    flat_idx = (idx + (jnp.arange(F, dtype=jnp.int32) * V)[None, :])
    gw = flat_idx.reshape(NW, N_PER_W)
    idxt = flat_idx.reshape(NW, BS, F).transpose(0, 2, 1)
    rows = _get_sc_gather()(emb2.reshape(F * V, E), idxt)
    e1 = _get_sc_gather_e1()(emb1.reshape(F * V), gw)
    out = _tc_compute(rows, Xv, e1.reshape(B, F), W1.T,
                      H.reshape(1, A), P.reshape(E, 1), bias.reshape(1, 1))
    return out.reshape(B)


# speedup vs baseline: 1.0009x; 1.0009x over previous
import functools

import jax
import jax.numpy as jnp
from jax import lax
from jax.experimental import pallas as pl
from jax.experimental.pallas import tpu as pltpu
from jax.experimental.pallas import tpu_sc as plsc

F = 26
V = 100000
E = 16
A = 16
B = 4096

NC = 2
NS = 16
NW = NC * NS
N = B * F
N_PER_W = N // NW
CH = 128
N_CH = N_PER_W // CH
BS = B // NW

BT = 256
FP = 32
EW = 512


@functools.lru_cache(maxsize=None)
def _get_sc_gather():
    mesh = plsc.VectorSubcoreMesh(core_axis_name="c", subcore_axis_name="s")

    @functools.partial(
        pl.kernel,
        mesh=mesh,
        compiler_params=pltpu.CompilerParams(use_tc_tiling_on_sc=False),
        out_type=jax.ShapeDtypeStruct((B, EW), jnp.float32),
        scratch_types=[
            pltpu.VMEM((F, BS), jnp.int32),
            pltpu.VMEM((F, BS, E), jnp.float32),
            pltpu.SemaphoreType.DMA,
            pltpu.SemaphoreType.DMA,
        ],
    )
    def _sc_gather(tbl_hbm, idx_hbm, rows_out, idx_v, buf, sem, sem2):
        wid = lax.axis_index("s") * NC + lax.axis_index("c")
        base = wid * BS
        pltpu.sync_copy(idx_hbm.at[wid], idx_v)
        cps = []
        for f in range(F):
            cps.append(pltpu.async_copy(
                tbl_hbm.at[idx_v.at[f]], buf.at[f], sem))
        outs = []
        for f in range(F):
            cps[f].wait()
            outs.append(pltpu.async_copy(
                buf.at[f],
                rows_out.at[pl.ds(base, BS), pl.ds(f * E, E)], sem2))
        for cp in outs:
            cp.wait()

    return _sc_gather


@functools.lru_cache(maxsize=None)
def _get_sc_gather_e1():
    mesh = plsc.VectorSubcoreMesh(core_axis_name="c", subcore_axis_name="s")

    @functools.partial(
        pl.kernel,
        mesh=mesh,
        compiler_params=pltpu.CompilerParams(use_tc_tiling_on_sc=False),
        out_type=jax.ShapeDtypeStruct((N,), jnp.float32),
        scratch_types=[
            pltpu.VMEM((N_PER_W,), jnp.int32),
            pltpu.VMEM((N_PER_W,), jnp.float32),
            pltpu.SemaphoreType.DMA,
        ],
    )
    def _sc_gather_e1(emb1_hbm, idx_hbm, e1_out, idx_v, e1_v, sem):
        wid = lax.axis_index("s") * NC + lax.axis_index("c")
        base = wid * N_PER_W
        pltpu.sync_copy(idx_hbm.at[wid], idx_v)
        pltpu.async_copy(emb1_hbm.at[idx_v], e1_v, sem).wait()
        pltpu.sync_copy(e1_v, e1_out.at[pl.ds(base, N_PER_W)])

    return _sc_gather_e1


def _tr(m):
    chunks = []
    L = m.shape[1]
    for c in range(0, L, 128):
        w = min(128, L - c)
        chunks.append(jnp.transpose(m[:, c:c + w]))
    return jnp.concatenate(chunks, axis=0) if len(chunks) > 1 else chunks[0]


def _tc_body(rows_ref, xvb_ref, e1b_ref, w1t_ref, h_ref, p_ref, b_ref, out_ref):
    w_col = jnp.sum(w1t_ref[...] * h_ref[...], axis=1, keepdims=True)
    p_col = p_ref[...]
    zpad = jnp.zeros((BT, FP - F), jnp.float32)
    xv = _tr(jnp.concatenate([xvb_ref[...], zpad], axis=1))
    e1t = _tr(jnp.concatenate([e1b_ref[...], zpad], axis=1))
    first = jnp.sum(e1t * xv, axis=0, keepdims=True)
    t = _tr(rows_ref[...][:, :F * E])
    s_all = jnp.concatenate(
        [t[f * E:(f + 1) * E, :] * xv[f:f + 1, :] for f in range(F)], axis=0)
    num = jnp.zeros_like(first)
    den = jnp.zeros_like(first)
    for i in range(F - 1):
        nj = F - 1 - i
        si = s_all[i * E:(i + 1) * E, :]
        swi = si * w_col
        spi = si * p_col
        rest = s_all[(i + 1) * E:, :]
        gw = jnp.sum((rest * jnp.tile(swi, (nj, 1))).reshape(nj, E, BT), axis=1)
        gp = jnp.sum((rest * jnp.tile(spi, (nj, 1))).reshape(nj, E, BT), axis=1)
        ew = jnp.exp(gw)
        den = den + jnp.sum(ew, axis=0, keepdims=True)
        num = num + jnp.sum(gp * ew, axis=0, keepdims=True)
    out_ref[...] = b_ref[...] + first + num / den


_tc_compute = pl.pallas_call(
    _tc_body,
    grid=(B // BT,),
    in_specs=[
        pl.BlockSpec((BT, EW), lambda i: (i, 0)),
        pl.BlockSpec((BT, F), lambda i: (i, 0)),
        pl.BlockSpec((BT, F), lambda i: (i, 0)),
        pl.BlockSpec((E, A), lambda i: (0, 0)),
        pl.BlockSpec((1, A), lambda i: (0, 0)),
        pl.BlockSpec((E, 1), lambda i: (0, 0)),
        pl.BlockSpec((1, 1), lambda i: (0, 0)),
    ],
    out_specs=pl.BlockSpec((1, BT), lambda i: (0, i)),
    out_shape=jax.ShapeDtypeStruct((1, B), jnp.float32),
)


def kernel(Xi, Xv, emb1, emb2, W1, b1, H, P, bias):
    del b1
    idx = Xi[:, :, 0].astype(jnp.int32)
    flat_idx = (idx + (jnp.arange(F, dtype=jnp.int32) * V)[None, :])
    gw = flat_idx.reshape(NW, N_PER_W)
    idxt = flat_idx.reshape(NW, BS, F).transpose(0, 2, 1)
    rows = _get_sc_gather()(emb2.reshape(F * V, E), idxt)
    e1 = _get_sc_gather_e1()(emb1.reshape(F * V), gw)
    out = _tc_compute(rows, Xv, e1.reshape(B, F), W1.T,
                      H.reshape(1, A), P.reshape(E, 1), bias.reshape(1, 1))
    return out.reshape(B)
